# baseline (speedup 1.0000x reference)
import functools

import jax
import jax.numpy as jnp
from jax import lax
from jax.experimental import pallas as pl
from jax.experimental.pallas import tpu as pltpu
from jax.experimental.pallas import tpu_sc as plsc

_VOCAB = 4096
_NT = 512


_W = 64


def _argmin_body(f_ref, emb2T_ref, idx_ref, losssum_ref):
    i = pl.program_id(0)
    f = f_ref[...]
    emb2T = emb2T_ref[...]
    mm2 = jnp.dot(f, emb2T)
    fs = jnp.sum(f * f, axis=1, keepdims=True)
    e = 0.5 * emb2T
    es = jnp.sum(e * e, axis=0, keepdims=True)

    mnv = (fs - mm2[:, :_W]) + es[:, :_W]
    mni = jnp.zeros(mnv.shape, jnp.float32)
    for c in range(1, _VOCAB // _W):
        dv = (fs - mm2[:, c * _W:(c + 1) * _W]) + es[:, c * _W:(c + 1) * _W]
        lt = dv < mnv
        mni = jnp.where(lt, jnp.float32(c), mni)
        mnv = jnp.minimum(mnv, dv)
    gmin = jnp.min(mnv, axis=1, keepdims=True)
    lane = lax.broadcasted_iota(jnp.int32, mnv.shape, 1).astype(jnp.float32)
    key = jnp.where(mnv == gmin, mni * jnp.float32(_W) + lane,
                    jnp.float32(_VOCAB))
    idx = jnp.min(key, axis=1).astype(jnp.int32)
    idx_ref[0, 0, :] = idx
    bs = jnp.sum(gmin).reshape(1, 1)

    @pl.when(i == 0)
    def _():
        losssum_ref[...] = bs

    @pl.when(i != 0)
    def _():
        losssum_ref[...] += bs


_CHUNK = 128


def _make_sc_gather(n_tokens, d_pad):
    info = plsc.get_sparse_core_info()
    nc, ns = info.num_cores, info.num_subcores
    nw = nc * ns
    b_per_w = n_tokens // nw
    n_chunks = b_per_w // _CHUNK
    mesh = plsc.VectorSubcoreMesh(core_axis_name="c", subcore_axis_name="s")

    @functools.partial(
        pl.kernel,
        mesh=mesh,
        out_type=jax.ShapeDtypeStruct((n_tokens, d_pad), jnp.float32),
        scratch_types=[
            pltpu.VMEM((n_chunks, _CHUNK), jnp.int32),
            pltpu.VMEM((b_per_w, d_pad), jnp.float32),
            pltpu.SemaphoreType.DMA,
        ],
        compiler_params=pltpu.CompilerParams(use_tc_tiling_on_sc=False),
    )
    def gather_k(table_hbm, idx_hbm, out_hbm, idx_v, rows_v, sem):
        wid = lax.axis_index("s") * nc + lax.axis_index("c")
        base = wid * b_per_w
        pltpu.sync_copy(idx_hbm.at[pl.ds(wid * n_chunks, n_chunks)], idx_v)
        copies = [
            pltpu.async_copy(
                table_hbm.at[idx_v.at[j]],
                rows_v.at[pl.ds(j * _CHUNK, _CHUNK)],
                sem,
            )
            for j in range(n_chunks)
        ]
        for c in copies:
            c.wait()
        pltpu.sync_copy(rows_v, out_hbm.at[pl.ds(base, b_per_w)])

    return gather_k


def kernel(feats, embedding):
    Bb, Ll, Dd = feats.shape
    n = Bb * Ll
    nb = n // _NT
    flat = feats.reshape(n, Dd)
    fpad = jnp.pad(flat, ((0, 0), (0, 8 - Dd)))
    emb2T = jnp.pad(embedding + embedding, ((0, 0), (0, 8 - Dd))).T

    idx3, losssum = pl.pallas_call(
        _argmin_body,
        grid=(nb,),
        in_specs=[
            pl.BlockSpec((_NT, 8), lambda i: (i, 0)),
            pl.BlockSpec((8, _VOCAB), lambda i: (0, 0)),
        ],
        out_specs=[
            pl.BlockSpec((1, 1, _NT), lambda i: (i, 0, 0)),
            pl.BlockSpec((1, 1), lambda i: (0, 0)),
        ],
        out_shape=[
            jax.ShapeDtypeStruct((nb, 1, _NT), jnp.int32),
            jax.ShapeDtypeStruct((1, 1), jnp.float32),
        ],
    )(fpad, emb2T)

    idx = idx3.reshape(n)
    loss = losssum[0, 0] * 1.25 / (n * Dd)

    d_pad = 16
    table = jnp.pad(embedding, ((0, 0), (0, d_pad - Dd)))
    idx2d = idx.reshape(n // _CHUNK, _CHUNK)
    quantp = _make_sc_gather(n, d_pad)(table, idx2d)
    quant = quantp[:, :Dd].reshape(Bb, Ll, Dd)
    quant_st = feats + (quant - feats)
    return (quant_st, idx.reshape(Bb, Ll), loss)


# speedup vs baseline: 2.6498x; 2.6498x over previous
import functools

import jax
import jax.numpy as jnp
from jax import lax
from jax.experimental import pallas as pl
from jax.experimental.pallas import tpu as pltpu
from jax.experimental.pallas import tpu_sc as plsc

_VOCAB = 4096
_NT = 512


_W = 128


def _argmin_body(f_ref, emb2T_ref, idx_ref, losssum_ref):
    i = pl.program_id(0)
    f = f_ref[...]
    emb2T = emb2T_ref[...]
    mm2 = jnp.dot(f, emb2T)
    fs = jnp.sum(f * f, axis=1, keepdims=True)
    e = 0.5 * emb2T
    es = jnp.sum(e * e, axis=0, keepdims=True)

    mnv = (fs - mm2[:, :_W]) + es[:, :_W]
    mni = jnp.zeros(mnv.shape, jnp.float32)
    for c in range(1, _VOCAB // _W):
        dv = (fs - mm2[:, c * _W:(c + 1) * _W]) + es[:, c * _W:(c + 1) * _W]
        lt = dv < mnv
        mni = jnp.where(lt, jnp.float32(c), mni)
        mnv = jnp.minimum(mnv, dv)
    gmin = jnp.min(mnv, axis=1, keepdims=True)
    lane = lax.broadcasted_iota(jnp.int32, mnv.shape, 1).astype(jnp.float32)
    key = jnp.where(mnv == gmin, mni * jnp.float32(_W) + lane,
                    jnp.float32(_VOCAB))
    idx = jnp.min(key, axis=1).astype(jnp.int32)
    idx_ref[0, 0, :] = idx
    bs = jnp.sum(gmin).reshape(1, 1)

    @pl.when(i == 0)
    def _():
        losssum_ref[...] = bs

    @pl.when(i != 0)
    def _():
        losssum_ref[...] += bs


_CHUNK = 128


def _make_sc_gather(n_tokens, d_pad):
    info = plsc.get_sparse_core_info()
    nc, ns = info.num_cores, info.num_subcores
    nw = nc * ns
    b_per_w = n_tokens // nw
    n_chunks = b_per_w // _CHUNK
    mesh = plsc.VectorSubcoreMesh(core_axis_name="c", subcore_axis_name="s")

    @functools.partial(
        pl.kernel,
        mesh=mesh,
        out_type=jax.ShapeDtypeStruct((n_tokens, d_pad), jnp.float32),
        scratch_types=[
            pltpu.VMEM((n_chunks, _CHUNK), jnp.int32),
            pltpu.VMEM((b_per_w, d_pad), jnp.float32),
            pltpu.SemaphoreType.DMA,
        ],
        compiler_params=pltpu.CompilerParams(use_tc_tiling_on_sc=False),
    )
    def gather_k(table_hbm, idx_hbm, out_hbm, idx_v, rows_v, sem):
        wid = lax.axis_index("s") * nc + lax.axis_index("c")
        base = wid * b_per_w
        pltpu.sync_copy(idx_hbm.at[pl.ds(wid * n_chunks, n_chunks)], idx_v)
        copies = [
            pltpu.async_copy(
                table_hbm.at[idx_v.at[j]],
                rows_v.at[pl.ds(j * _CHUNK, _CHUNK)],
                sem,
            )
            for j in range(n_chunks)
        ]
        for c in copies:
            c.wait()
        pltpu.sync_copy(rows_v, out_hbm.at[pl.ds(base, b_per_w)])

    return gather_k


def kernel(feats, embedding):
    Bb, Ll, Dd = feats.shape
    n = Bb * Ll
    nb = n // _NT
    flat = feats.reshape(n, Dd)
    fpad = jnp.pad(flat, ((0, 0), (0, 8 - Dd)))
    emb2T = jnp.pad(embedding + embedding, ((0, 0), (0, 8 - Dd))).T

    idx3, losssum = pl.pallas_call(
        _argmin_body,
        grid=(nb,),
        in_specs=[
            pl.BlockSpec((_NT, 8), lambda i: (i, 0)),
            pl.BlockSpec((8, _VOCAB), lambda i: (0, 0)),
        ],
        out_specs=[
            pl.BlockSpec((1, 1, _NT), lambda i: (i, 0, 0)),
            pl.BlockSpec((1, 1), lambda i: (0, 0)),
        ],
        out_shape=[
            jax.ShapeDtypeStruct((nb, 1, _NT), jnp.int32),
            jax.ShapeDtypeStruct((1, 1), jnp.float32),
        ],
    )(fpad, emb2T)

    idx = idx3.reshape(n)
    loss = losssum[0, 0] * 1.25 / (n * Dd)

    d_pad = 16
    table = jnp.pad(embedding, ((0, 0), (0, d_pad - Dd)))
    idx2d = idx.reshape(n // _CHUNK, _CHUNK)
    quantp = _make_sc_gather(n, d_pad)(table, idx2d)
    quant = quantp[:, :Dd].reshape(Bb, Ll, Dd)
    quant_st = feats + (quant - feats)
    return (quant_st, idx.reshape(Bb, Ll), loss)


# speedup vs baseline: 2.7955x; 1.0550x over previous
import functools

import jax
import jax.numpy as jnp
from jax import lax
from jax.experimental import pallas as pl
from jax.experimental.pallas import tpu as pltpu
from jax.experimental.pallas import tpu_sc as plsc

_VOCAB = 4096
_NT = 512


_W = 128


def _argmin_body(f_ref, emb2T_ref, idx_ref, losssum_ref):
    i = pl.program_id(0)
    f = f_ref[...]
    emb2T = emb2T_ref[...]
    mm2 = jnp.dot(f, emb2T)
    fs = jnp.sum(f * f, axis=1, keepdims=True)
    e = 0.5 * emb2T
    es = jnp.sum(e * e, axis=0, keepdims=True)

    mnv = (fs - mm2[:, :_W]) + es[:, :_W]
    mni = jnp.zeros(mnv.shape, jnp.float32)
    for c in range(1, _VOCAB // _W):
        dv = (fs - mm2[:, c * _W:(c + 1) * _W]) + es[:, c * _W:(c + 1) * _W]
        lt = dv < mnv
        mni = jnp.where(lt, jnp.float32(c), mni)
        mnv = jnp.minimum(mnv, dv)
    gmin = jnp.min(mnv, axis=1, keepdims=True)
    lane = lax.broadcasted_iota(jnp.int32, mnv.shape, 1).astype(jnp.float32)
    key = jnp.where(mnv == gmin, mni * jnp.float32(_W) + lane,
                    jnp.float32(_VOCAB))
    idx = jnp.min(key, axis=1).astype(jnp.int32)
    idx_ref[0, 0, :] = idx
    bs = jnp.sum(gmin).reshape(1, 1)

    @pl.when(i == 0)
    def _():
        losssum_ref[...] = bs

    @pl.when(i != 0)
    def _():
        losssum_ref[...] += bs


_CHUNK = 128


def _make_sc_gather(n_tokens, d_pad):
    info = plsc.get_sparse_core_info()
    nc, ns = info.num_cores, info.num_subcores
    nw = nc * ns
    b_per_w = n_tokens // nw
    n_chunks = b_per_w // _CHUNK
    mesh = plsc.VectorSubcoreMesh(core_axis_name="c", subcore_axis_name="s")

    @functools.partial(
        pl.kernel,
        mesh=mesh,
        out_type=jax.ShapeDtypeStruct((n_tokens, d_pad), jnp.float32),
        scratch_types=[
            pltpu.VMEM((n_chunks, _CHUNK), jnp.int32),
            pltpu.VMEM((b_per_w, d_pad), jnp.float32),
            pltpu.SemaphoreType.DMA,
        ],
        compiler_params=pltpu.CompilerParams(use_tc_tiling_on_sc=False),
    )
    def gather_k(table_hbm, idx_hbm, out_hbm, idx_v, rows_v, sem):
        wid = lax.axis_index("s") * nc + lax.axis_index("c")
        base = wid * b_per_w
        pltpu.sync_copy(idx_hbm.at[pl.ds(wid * n_chunks, n_chunks)], idx_v)
        copies = [
            pltpu.async_copy(
                table_hbm.at[idx_v.at[j]],
                rows_v.at[pl.ds(j * _CHUNK, _CHUNK)],
                sem,
            )
            for j in range(n_chunks)
        ]
        for c in copies:
            c.wait()
        pltpu.sync_copy(rows_v, out_hbm.at[pl.ds(base, b_per_w)])

    return gather_k


def _argmin_call(fpad_half, emb2T):
    nh = fpad_half.shape[0]
    nb = nh // _NT
    return pl.pallas_call(
        _argmin_body,
        grid=(nb,),
        in_specs=[
            pl.BlockSpec((_NT, 8), lambda i: (i, 0)),
            pl.BlockSpec((8, _VOCAB), lambda i: (0, 0)),
        ],
        out_specs=[
            pl.BlockSpec((1, 1, _NT), lambda i: (i, 0, 0)),
            pl.BlockSpec((1, 1), lambda i: (0, 0)),
        ],
        out_shape=[
            jax.ShapeDtypeStruct((nb, 1, _NT), jnp.int32),
            jax.ShapeDtypeStruct((1, 1), jnp.float32),
        ],
    )(fpad_half, emb2T)


def kernel(feats, embedding):
    Bb, Ll, Dd = feats.shape
    n = Bb * Ll
    nh = n // 2
    flat = feats.reshape(n, Dd)
    fpad = jnp.pad(flat, ((0, 0), (0, 8 - Dd)))
    emb2T = jnp.pad(embedding + embedding, ((0, 0), (0, 8 - Dd))).T

    d_pad = 16
    table = jnp.pad(embedding, ((0, 0), (0, d_pad - Dd)))
    gather = _make_sc_gather(nh, d_pad)

    idx3_a, loss_a = _argmin_call(fpad[:nh], emb2T)
    idx_a = idx3_a.reshape(nh)
    quant_a = gather(table, idx_a.reshape(nh // _CHUNK, _CHUNK))
    idx3_b, loss_b = _argmin_call(fpad[nh:], emb2T)
    idx_b = idx3_b.reshape(nh)
    quant_b = gather(table, idx_b.reshape(nh // _CHUNK, _CHUNK))

    idx = jnp.concatenate([idx_a, idx_b])
    loss = (loss_a[0, 0] + loss_b[0, 0]) * 1.25 / (n * Dd)
    quantp = jnp.concatenate([quant_a, quant_b])
    quant_st = quantp[:, :Dd].reshape(Bb, Ll, Dd)
    return (quant_st, idx.reshape(Bb, Ll), loss)
